# baseline (speedup 1.0000x reference)
import jax
import jax.numpy as jnp
from jax import lax
from jax.experimental import pallas as pl
from jax.experimental.pallas import tpu as pltpu
from jax.experimental.pallas import tpu_sc as plsc


def _router_body(x_ref, gw_ref, wab_ref, i12_ref):
    x = x_ref[...].astype(jnp.bfloat16)
    gw = gw_ref[...].astype(jnp.bfloat16)
    logits = lax.dot_general(
        x, gw, (((1,), (1,)), ((), ())),
        preferred_element_type=jnp.float32)
    E = logits.shape[1]
    eidx = lax.broadcasted_iota(jnp.int32, logits.shape, 1)
    m1 = jnp.max(logits, axis=1, keepdims=True)
    i1 = jnp.min(jnp.where(logits == m1, eidx, E), axis=1, keepdims=True)
    neg = jnp.float32(jnp.finfo(jnp.float32).min)
    masked = jnp.where(eidx == i1, neg, logits)
    m2 = jnp.max(masked, axis=1, keepdims=True)
    i2 = jnp.min(jnp.where(masked == m2, eidx, E), axis=1, keepdims=True)
    wa = jax.nn.sigmoid(m1 - m2)
    wb = jax.nn.sigmoid(m2 - m1)
    wab_ref[...] = jnp.concatenate([wa, wb], axis=1)
    i12_ref[...] = jnp.concatenate([i1, i2], axis=1)


def _make_scan_body(E, RB, NBLK, TK, NT, TMG):
    def body(e_ref, p_ref, posi_ref, te_ref, carry_ref):
        b = pl.program_id(0)

        @pl.when(b == 0)
        def _init():
            carry_ref[...] = jnp.zeros_like(carry_ref)

        @pl.when(b < NBLK)
        def _scan_step():
            ef = e_ref[pl.ds(b * RB, RB), :]
            lane = lax.broadcasted_iota(jnp.int32, (RB, E), 1)
            C = (ef == lane).astype(jnp.float32)
            r = lax.broadcasted_iota(jnp.int32, (RB, RB), 0)
            c = lax.broadcasted_iota(jnp.int32, (RB, RB), 1)
            tril = (r > c).astype(jnp.float32)
            ranks = lax.dot_general(
                tril, C, (((1,), (0,)), ((), ())),
                preferred_element_type=jnp.float32)
            p_ref[pl.ds(b * RB, RB), :] = ranks + carry_ref[...]
            carry_ref[...] += jnp.sum(C, axis=0, keepdims=True)

        @pl.when(b == NBLK)
        def _finalize():
            counts = carry_ref[...]
            tiles = jnp.floor((counts + (TMG - 1)) / TMG)
            r8 = lax.broadcasted_iota(jnp.int32, (E, E), 0)
            c8 = lax.broadcasted_iota(jnp.int32, (E, E), 1)
            upper = (r8 < c8).astype(jnp.float32)
            cumt = lax.dot_general(
                tiles, upper, (((1,), (0,)), ((), ())),
                preferred_element_type=jnp.float32)
            padded_off = cumt * TMG
            ef = e_ref[...]
            lane = lax.broadcasted_iota(jnp.int32, (TK, E), 1)
            C = (ef == lane).astype(jnp.float32)
            base = lax.dot_general(
                C, padded_off, (((1,), (1,)), ((), ())),
                preferred_element_type=jnp.float32)
            rank = jnp.sum(p_ref[...] * C, axis=1, keepdims=True)
            posi_ref[...] = (base + rank).astype(jnp.int32)
            jrow = lax.broadcasted_iota(jnp.int32, (NT, E), 0).astype(jnp.float32)
            te = jnp.sum((jrow >= cumt).astype(jnp.float32), axis=1,
                         keepdims=True) - 1.0
            te_ref[...] = te.astype(jnp.int32)
    return body


def _sc_dispatch(x_flat, pos1, pos2, NP):
    T, H = x_flat.shape
    info = plsc.get_sparse_core_info()
    NW = info.num_cores * info.num_subcores
    chunk = T // NW
    mesh = plsc.VectorSubcoreMesh(core_axis_name="c", subcore_axis_name="s")

    def body(x_hbm, p1_hbm, p2_hbm, xs_hbm, xbuf, idx1, idx2, sem):
        wid = lax.axis_index("s") * info.num_cores + lax.axis_index("c")
        base = wid * chunk
        pltpu.sync_copy(x_hbm.at[pl.ds(base, chunk)], xbuf)
        pltpu.sync_copy(p1_hbm.at[pl.ds(base, chunk)], idx1)
        pltpu.sync_copy(p2_hbm.at[pl.ds(base, chunk)], idx2)
        pltpu.async_copy(xbuf, xs_hbm.at[idx1], sem).wait()
        pltpu.async_copy(xbuf, xs_hbm.at[idx2], sem).wait()

    return pl.kernel(
        body,
        out_type=jax.ShapeDtypeStruct((NP, H), jnp.float32),
        mesh=mesh,
        scratch_types=[
            pltpu.VMEM((chunk, H), jnp.float32),
            pltpu.VMEM((chunk,), jnp.int32),
            pltpu.VMEM((chunk,), jnp.int32),
            pltpu.SemaphoreType.DMA,
        ],
    )(x_flat, pos1, pos2)


def _sc_return(y_sorted, pos1, pos2):
    NP, H = y_sorted.shape
    T = pos1.shape[0]
    info = plsc.get_sparse_core_info()
    NW = info.num_cores * info.num_subcores
    chunk = T // NW
    mesh = plsc.VectorSubcoreMesh(core_axis_name="c", subcore_axis_name="s")

    def body(y_hbm, p1_hbm, p2_hbm, z_hbm, buf, idx, sem):
        wid = lax.axis_index("s") * info.num_cores + lax.axis_index("c")
        base = wid * chunk
        pltpu.sync_copy(p1_hbm.at[pl.ds(base, chunk)], idx)
        pltpu.async_copy(y_hbm.at[idx], buf, sem).wait()
        pltpu.sync_copy(buf, z_hbm.at[0, pl.ds(base, chunk)])
        pltpu.sync_copy(p2_hbm.at[pl.ds(base, chunk)], idx)
        pltpu.async_copy(y_hbm.at[idx], buf, sem).wait()
        pltpu.sync_copy(buf, z_hbm.at[1, pl.ds(base, chunk)])

    return pl.kernel(
        body,
        out_type=jax.ShapeDtypeStruct((2, T, H), jnp.float32),
        mesh=mesh,
        scratch_types=[
            pltpu.VMEM((chunk, H), jnp.float32),
            pltpu.VMEM((chunk,), jnp.int32),
            pltpu.SemaphoreType.DMA,
        ],
    )(y_sorted, pos1, pos2)


def _make_ffn_body(TMG):
    def body(te_ref, xs_ref, w1_ref, w2_ref, out_ref):
        f = pl.program_id(0)
        t = pl.program_id(1)
        rows = pl.ds(t * TMG, TMG)
        x = xs_ref[rows, :].astype(jnp.bfloat16)
        w1b = w1_ref[0].astype(jnp.bfloat16)
        h = lax.dot_general(
            x, w1b, (((1,), (1,)), ((), ())),
            preferred_element_type=jnp.float32)
        h = h * jax.nn.sigmoid(h)
        hb = h.astype(jnp.bfloat16)
        w2b = w2_ref[0].astype(jnp.bfloat16)
        y = lax.dot_general(
            hb, w2b, (((1,), (1,)), ((), ())),
            preferred_element_type=jnp.float32)

        @pl.when(f == 0)
        def _set():
            out_ref[rows, :] = y

        @pl.when(f > 0)
        def _acc():
            out_ref[rows, :] += y
    return body


def _combine_body(z_ref, wab_ref, o_ref):
    za = z_ref[0]
    zb = z_ref[1]
    wa = wab_ref[:, 0:1]
    wb = wab_ref[:, 1:2]
    o_ref[...] = za * wa + zb * wb


def kernel(x, gate_w, w1, w2):
    B, S, H = x.shape
    E, F, _ = w1.shape
    T = B * S
    K = 2
    TK = T * K
    TMG = 128 if TK >= 1024 else 16
    NT = TK // TMG + E
    NP = NT * TMG
    x_flat = x.reshape(T, H)

    wab, i12 = pl.pallas_call(
        _router_body,
        grid=(1,),
        in_specs=[
            pl.BlockSpec((T, H), lambda i: (0, 0)),
            pl.BlockSpec((E, H), lambda i: (0, 0)),
        ],
        out_specs=[
            pl.BlockSpec((T, K), lambda i: (0, 0)),
            pl.BlockSpec((T, K), lambda i: (0, 0)),
        ],
        out_shape=[
            jax.ShapeDtypeStruct((T, K), jnp.float32),
            jax.ShapeDtypeStruct((T, K), jnp.int32),
        ],
    )(x_flat, gate_w)

    e_flat = i12.reshape(TK, 1)
    RB = min(512, TK)
    NBLK = TK // RB

    _, posi, te = pl.pallas_call(
        _make_scan_body(E, RB, NBLK, TK, NT, TMG),
        grid=(NBLK + 1,),
        in_specs=[pl.BlockSpec((TK, 1), lambda b: (0, 0))],
        out_specs=[
            pl.BlockSpec((TK, E), lambda b: (0, 0)),
            pl.BlockSpec((TK, 1), lambda b: (0, 0)),
            pl.BlockSpec((NT, 1), lambda b: (0, 0)),
        ],
        out_shape=[
            jax.ShapeDtypeStruct((TK, E), jnp.float32),
            jax.ShapeDtypeStruct((TK, 1), jnp.int32),
            jax.ShapeDtypeStruct((NT, 1), jnp.int32),
        ],
        scratch_shapes=[pltpu.VMEM((1, E), jnp.float32)],
    )(e_flat)

    posi_flat = posi.reshape(TK)
    pos1 = posi_flat[0::2]
    pos2 = posi_flat[1::2]
    te_flat = te.reshape(NT)

    x_sorted = _sc_dispatch(x_flat, pos1, pos2, NP)

    TF = min(512, F)
    grid_spec = pltpu.PrefetchScalarGridSpec(
        num_scalar_prefetch=1,
        grid=(F // TF, NT),
        in_specs=[
            pl.BlockSpec((NP, H), lambda f, t, te_r: (0, 0)),
            pl.BlockSpec((1, TF, H), lambda f, t, te_r: (te_r[t], f, 0)),
            pl.BlockSpec((1, H, TF), lambda f, t, te_r: (te_r[t], 0, f)),
        ],
        out_specs=pl.BlockSpec((NP, H), lambda f, t, te_r: (0, 0)),
    )
    y_sorted = pl.pallas_call(
        _make_ffn_body(TMG),
        grid_spec=grid_spec,
        out_shape=jax.ShapeDtypeStruct((NP, H), jnp.float32),
    )(te_flat, x_sorted, w1, w2)

    z = _sc_return(y_sorted, pos1, pos2)

    BT = min(512, T)
    out = pl.pallas_call(
        _combine_body,
        grid=(T // BT,),
        in_specs=[
            pl.BlockSpec((K, BT, H), lambda i: (0, i, 0)),
            pl.BlockSpec((BT, K), lambda i: (i, 0)),
        ],
        out_specs=pl.BlockSpec((BT, H), lambda i: (i, 0)),
        out_shape=jax.ShapeDtypeStruct((T, H), jnp.float32),
    )(z, wab)

    return out.reshape(B, S, H)
